# baseline (speedup 1.0000x reference)
import functools

import jax
import jax.numpy as jnp
from jax import lax
from jax.experimental import pallas as pl
from jax.experimental.pallas import tpu as pltpu
from jax.experimental.pallas import tpu_sc as plsc

_NCG = 4


def _tc_body(up_pts_ref, up_feat_ref, down_ptsT_ref, down_feat_ref,
             w_up_ref, b_up_ref, w_down_ref, b_down_ref,
             *out_refs):
    base_refs = out_refs[0:_NCG]
    idx_ref = out_refs[_NCG]
    w_ref = out_refs[_NCG + 1]
    downf_refs = out_refs[_NCG + 2:]
    n = down_ptsT_ref.shape[1]
    bm = up_pts_ref.shape[0]
    out_c = w_up_ref.shape[1]
    cw = out_c // _NCG

    @pl.when(pl.program_id(0) == 0)
    def _():
        down_f = (jnp.dot(down_feat_ref[...], w_down_ref[...],
                          preferred_element_type=jnp.float32)
                  + b_down_ref[...])
        for g in range(_NCG):
            downf_refs[g][...] = down_f[:, g * cw:(g + 1) * cw]

    dist = jnp.zeros((bm, n), dtype=jnp.float32)
    for c in range(3):
        u = up_pts_ref[:, c:c + 1]
        d = down_ptsT_ref[c:c + 1, :]
        diff = u - d
        dist = diff * diff + dist

    colf = lax.broadcasted_iota(jnp.int32, (bm, n), 1).astype(jnp.float32)
    nf = jnp.float32(n)
    big = jnp.float32(jnp.inf)

    iks = []
    vals = []
    d_work = dist
    for k in range(3):
        mk = jnp.min(d_work, axis=1, keepdims=True)
        sel = d_work == mk
        ik = jnp.min(jnp.where(sel, colf, nf), axis=1, keepdims=True)
        iks.append(ik)
        vals.append(mk)
        if k < 2:
            d_work = jnp.where(sel, big, d_work)

    r = [1.0 / (v + 1e-8) for v in vals]
    denom = r[0] + r[1] + r[2]
    w = [ri / denom for ri in r]

    idx_ref[...] = jnp.concatenate(iks, axis=1).astype(jnp.int32)
    w_ref[...] = jnp.concatenate(w, axis=1)
    base = (jnp.dot(up_feat_ref[...], w_up_ref[...],
                    preferred_element_type=jnp.float32)
            + b_up_ref[...])
    for g in range(_NCG):
        base_refs[g][...] = base[:, g * cw:(g + 1) * cw]


def _tc_stage(up_points, up_features, down_ptsT, down_features,
              W_up, b_up2, W_down, b_down2, bm):
    m = up_points.shape[0]
    n = down_ptsT.shape[1]
    up_c = up_features.shape[1]
    down_c = down_features.shape[1]
    out_c = W_up.shape[1]
    cw = out_c // _NCG
    grid = (m // bm,)
    return pl.pallas_call(
        _tc_body,
        grid=grid,
        in_specs=[
            pl.BlockSpec((bm, 3), lambda i: (i, 0)),
            pl.BlockSpec((bm, up_c), lambda i: (i, 0)),
            pl.BlockSpec((3, n), lambda i: (0, 0)),
            pl.BlockSpec((n, down_c), lambda i: (0, 0)),
            pl.BlockSpec((up_c, out_c), lambda i: (0, 0)),
            pl.BlockSpec((1, out_c), lambda i: (0, 0)),
            pl.BlockSpec((down_c, out_c), lambda i: (0, 0)),
            pl.BlockSpec((1, out_c), lambda i: (0, 0)),
        ],
        out_specs=(
            [pl.BlockSpec((bm, cw), lambda i: (i, 0)) for _ in range(_NCG)]
            + [pl.BlockSpec((bm, 3), lambda i: (i, 0))] * 2
            + [pl.BlockSpec((n, cw), lambda i: (0, 0)) for _ in range(_NCG)]
        ),
        out_shape=(
            [jax.ShapeDtypeStruct((m, cw), jnp.float32)
             for _ in range(_NCG)]
            + [jax.ShapeDtypeStruct((m, 3), jnp.int32),
               jax.ShapeDtypeStruct((m, 3), jnp.float32)]
            + [jax.ShapeDtypeStruct((n, cw), jnp.float32)
             for _ in range(_NCG)]
        ),
    )(up_points, up_features, down_ptsT, down_features,
      W_up, b_up2, W_down, b_down2)


def _sc_interp(idx_flat, w_flat, bases_flat, downfs_flat, m, n, cw):
    nmc = 32 // _NCG
    mc = m // nmc

    mesh = plsc.VectorSubcoreMesh(core_axis_name="c", subcore_axis_name="s")

    @functools.partial(
        pl.kernel, mesh=mesh,
        compiler_params=pltpu.CompilerParams(needs_layout_passes=False),
        out_type=[jax.ShapeDtypeStruct((m * cw,), jnp.float32)
                  for _ in range(_NCG)],
        scratch_types=[
            pltpu.VMEM((mc * 3,), jnp.int32),
            pltpu.VMEM((mc * 3,), jnp.float32),
            pltpu.VMEM((n * cw,), jnp.float32),
            pltpu.VMEM((mc * cw,), jnp.float32),
        ],
    )
    def sc_k(idx_hbm, w_hbm,
             base0, base1, base2, base3,
             downf0, downf1, downf2, downf3,
             out0, out1, out2, out3,
             idx_v, w_v, table_v, acc_v):
        base_hbms = [base0, base1, base2, base3]
        downf_hbms = [downf0, downf1, downf2, downf3]
        out_hbms = [out0, out1, out2, out3]
        wid = lax.axis_index("s") * 2 + lax.axis_index("c")
        mchunk = wid // _NCG
        pbase = mchunk * mc

        pltpu.sync_copy(idx_hbm.at[pl.ds(pbase * 3, mc * 3)], idx_v)
        pltpu.sync_copy(w_hbm.at[pl.ds(pbase * 3, mc * 3)], w_v)

        def body(g, carry):
            lane = lax.broadcasted_iota(jnp.int32, (16,), 0)
            rowv = g * 16 + lane
            rows3 = rowv * 3
            rowsc = rowv * cw
            ivs = []
            wvs = []
            for k in range(3):
                ivs.append(plsc.load_gather(idx_v, [rows3 + k]))
                wvs.append(plsc.load_gather(w_v, [rows3 + k]))
            tidx = [iv * cw for iv in ivs]
            for c in range(cw):
                f0 = plsc.load_gather(table_v, [tidx[0] + c])
                f1 = plsc.load_gather(table_v, [tidx[1] + c])
                f2 = plsc.load_gather(table_v, [tidx[2] + c])
                contrib = wvs[0] * f0 + wvs[1] * f1 + wvs[2] * f2
                plsc.addupdate_scatter(acc_v, [rowsc + c], contrib)
            return carry

        for cg in range(_NCG):
            @pl.when(wid % _NCG == cg)
            def _():
                pltpu.sync_copy(downf_hbms[cg], table_v)
                pltpu.sync_copy(base_hbms[cg].at[pl.ds(pbase * cw, mc * cw)],
                                acc_v)
                lax.fori_loop(0, mc // 16, body, 0)
                pltpu.sync_copy(acc_v,
                                out_hbms[cg].at[pl.ds(pbase * cw, mc * cw)])

    outs = sc_k(idx_flat, w_flat, *bases_flat, *downfs_flat)
    return jnp.concatenate(
        [o.reshape(m, cw) for o in outs], axis=1)


def kernel(up_points, up_features, down_points, down_features,
           W_up, b_up, W_down, b_down):
    out_c = W_up.shape[1]
    down_ptsT = down_points.T
    b_up2 = b_up.reshape(1, out_c)
    b_down2 = b_down.reshape(1, out_c)
    outs = _tc_stage(up_points, up_features, down_ptsT, down_features,
                     W_up, b_up2, W_down, b_down2, bm=2048)
    bases = outs[0:_NCG]
    idx = outs[_NCG]
    w = outs[_NCG + 1]
    downfs = outs[_NCG + 2:]
    m = idx.shape[0]
    n = down_points.shape[0]
    cw = out_c // _NCG
    return _sc_interp(idx.reshape(m * 3), w.reshape(m * 3),
                      [b.reshape(m * cw) for b in bases],
                      [d.reshape(n * cw) for d in downfs],
                      m, n, cw)


# speedup vs baseline: 1.8438x; 1.8438x over previous
import functools

import jax
import jax.numpy as jnp
from jax import lax
from jax.experimental import pallas as pl
from jax.experimental.pallas import tpu as pltpu
from jax.experimental.pallas import tpu_sc as plsc

_NCG = 4


def _tc_body(up_pts_ref, up_feat_ref, down_ptsT_ref, down_feat_ref,
             w_up_ref, b_up_ref, w_down_ref, b_down_ref,
             baseT_ref, idx_ref, w_ref, downfT_ref):
    n = down_ptsT_ref.shape[1]
    bm = up_pts_ref.shape[0]

    @pl.when(pl.program_id(0) == 0)
    def _():
        downfT_ref[...] = (
            lax.dot_general(w_down_ref[...], down_feat_ref[...],
                            (((0,), (1,)), ((), ())),
                            preferred_element_type=jnp.float32)
            + b_down_ref[...]
        )

    dist = jnp.zeros((bm, n), dtype=jnp.float32)
    for c in range(3):
        u = up_pts_ref[:, c:c + 1]
        d = down_ptsT_ref[c:c + 1, :]
        diff = u - d
        dist = diff * diff + dist

    colf = lax.broadcasted_iota(jnp.int32, (bm, n), 1).astype(jnp.float32)
    nf = jnp.float32(n)
    big = jnp.float32(jnp.inf)

    iks = []
    vals = []
    d_work = dist
    for k in range(3):
        mk = jnp.min(d_work, axis=1, keepdims=True)
        sel = d_work == mk
        ik = jnp.min(jnp.where(sel, colf, nf), axis=1, keepdims=True)
        iks.append(ik)
        vals.append(mk)
        if k < 2:
            d_work = jnp.where(sel, big, d_work)

    r = [1.0 / (v + 1e-8) for v in vals]
    denom = r[0] + r[1] + r[2]
    w = [ri / denom for ri in r]

    idx_ref[...] = jnp.concatenate(iks, axis=1).astype(jnp.int32)
    w_ref[...] = jnp.concatenate(w, axis=1)
    baseT_ref[...] = (
        lax.dot_general(w_up_ref[...], up_feat_ref[...],
                        (((0,), (1,)), ((), ())),
                        preferred_element_type=jnp.float32)
        + b_up_ref[...]
    )


def _tc_stage(up_points, up_features, down_ptsT, down_features,
              W_up, b_up_col, W_down, b_down_col, bm):
    m = up_points.shape[0]
    n = down_ptsT.shape[1]
    up_c = up_features.shape[1]
    down_c = down_features.shape[1]
    out_c = W_up.shape[1]
    grid = (m // bm,)
    return pl.pallas_call(
        _tc_body,
        grid=grid,
        in_specs=[
            pl.BlockSpec((bm, 3), lambda i: (i, 0)),
            pl.BlockSpec((bm, up_c), lambda i: (i, 0)),
            pl.BlockSpec((3, n), lambda i: (0, 0)),
            pl.BlockSpec((n, down_c), lambda i: (0, 0)),
            pl.BlockSpec((up_c, out_c), lambda i: (0, 0)),
            pl.BlockSpec((out_c, 1), lambda i: (0, 0)),
            pl.BlockSpec((down_c, out_c), lambda i: (0, 0)),
            pl.BlockSpec((out_c, 1), lambda i: (0, 0)),
        ],
        out_specs=[
            pl.BlockSpec((out_c, bm), lambda i: (0, i)),
            pl.BlockSpec((bm, 3), lambda i: (i, 0)),
            pl.BlockSpec((bm, 3), lambda i: (i, 0)),
            pl.BlockSpec((out_c, n), lambda i: (0, 0)),
        ],
        out_shape=[
            jax.ShapeDtypeStruct((out_c, m), jnp.float32),
            jax.ShapeDtypeStruct((m, 3), jnp.int32),
            jax.ShapeDtypeStruct((m, 3), jnp.float32),
            jax.ShapeDtypeStruct((out_c, n), jnp.float32),
        ],
    )(up_points, up_features, down_ptsT, down_features,
      W_up, b_up_col, W_down, b_down_col)


def _sc_interp(idx_flat, w_flat, baseT, downfT):
    out_c, m = baseT.shape
    n = downfT.shape[1]
    cw = out_c // _NCG
    nmc = 32 // _NCG
    mc = m // nmc

    mesh = plsc.VectorSubcoreMesh(core_axis_name="c", subcore_axis_name="s")

    @functools.partial(
        pl.kernel, mesh=mesh,
        compiler_params=pltpu.CompilerParams(needs_layout_passes=False),
        out_type=jax.ShapeDtypeStruct((out_c, m), jnp.float32),
        scratch_types=[
            pltpu.VMEM((mc * 3,), jnp.int32),
            pltpu.VMEM((mc * 3,), jnp.float32),
            pltpu.VMEM((cw, n), jnp.float32),
            pltpu.VMEM((cw, mc), jnp.float32),
        ],
    )
    def sc_k(idx_hbm, w_hbm, baseT_hbm, downfT_hbm, outT_hbm,
             idx_v, w_v, table_v, acc_v):
        wid = lax.axis_index("s") * 2 + lax.axis_index("c")
        mchunk = wid // _NCG
        cg = wid % _NCG
        pbase = pl.multiple_of(mchunk * mc, 128)
        cbase = pl.multiple_of(cg * cw, 8)

        pltpu.sync_copy(idx_hbm.at[pl.ds(pbase * 3, mc * 3)], idx_v)
        pltpu.sync_copy(w_hbm.at[pl.ds(pbase * 3, mc * 3)], w_v)
        pltpu.sync_copy(downfT_hbm.at[pl.ds(cbase, cw), :], table_v)
        pltpu.sync_copy(baseT_hbm.at[pl.ds(cbase, cw), pl.ds(pbase, mc)],
                        acc_v)

        def body(g, carry):
            lane = lax.broadcasted_iota(jnp.int32, (16,), 0)
            rowv = g * 16 + lane
            rows3 = rowv * 3
            ivs = []
            wvs = []
            for k in range(3):
                ivs.append(plsc.load_gather(idx_v, [rows3 + k]))
                wvs.append(plsc.load_gather(w_v, [rows3 + k]))
            for c in range(cw):
                cc = jnp.full((16,), c, jnp.int32)
                f0 = plsc.load_gather(table_v, [cc, ivs[0]])
                f1 = plsc.load_gather(table_v, [cc, ivs[1]])
                f2 = plsc.load_gather(table_v, [cc, ivs[2]])
                contrib = wvs[0] * f0 + wvs[1] * f1 + wvs[2] * f2
                plsc.addupdate_scatter(acc_v, [cc, rowv], contrib)
            return carry

        lax.fori_loop(0, mc // 16, body, 0)

        pltpu.sync_copy(acc_v,
                        outT_hbm.at[pl.ds(cbase, cw), pl.ds(pbase, mc)])

    return sc_k(idx_flat, w_flat, baseT, downfT)


def kernel(up_points, up_features, down_points, down_features,
           W_up, b_up, W_down, b_down):
    m = up_points.shape[0]
    out_c = W_up.shape[1]
    down_ptsT = down_points.T
    b_up_col = b_up.reshape(out_c, 1)
    b_down_col = b_down.reshape(out_c, 1)
    baseT, idx, w, downfT = _tc_stage(
        up_points, up_features, down_ptsT, down_features,
        W_up, b_up_col, W_down, b_down_col, bm=2048)
    outT = _sc_interp(idx.reshape(m * 3), w.reshape(m * 3), baseT, downfT)
    return outT.T
